# baseline (speedup 1.0000x reference)
import functools

import jax
import jax.numpy as jnp
from jax import lax
from jax.experimental import pallas as pl
from jax.experimental.pallas import tpu as pltpu
from jax.experimental.pallas import tpu_sc as plsc

N_ATOMS = 50000
N_EDGES = 800000
N_PAIRS = 3200000
F_DIM = 92
F_PAD = 96
R_PAD = 8

NC, NS, LANES = 2, 16, 16
NW = NC * NS

CA = 512
CE = 1024
CP = 2048


def _cdiv(a, b):
    return (a + b - 1) // b


NCH_A = _cdiv(N_ATOMS, CA)
NCH_E = _cdiv(N_EDGES, CE)
NCH_P = _cdiv(N_PAIRS, CP)

_MESH = plsc.VectorSubcoreMesh(
    core_axis_name="c", subcore_axis_name="s", num_cores=NC, num_subcores=NS
)
_PARAMS = pltpu.CompilerParams(
    needs_layout_passes=False, use_tc_tiling_on_sc=False
)


def _wid():
    return lax.axis_index("s") * NC + lax.axis_index("c")


def _my_chunks(total_chunks, wid):
    return (total_chunks - 1 - wid) // NW + 1


def _rsqrt(q):
    i = plsc.bitcast(q, jnp.int32)
    i = 0x5F3759DF - lax.shift_right_logical(i, 1)
    y = plsc.bitcast(i, jnp.float32)
    for _ in range(2):
        y = y * (1.5 - 0.5 * q * y * y)
    return y


@functools.partial(
    pl.kernel,
    out_type=(
        jax.ShapeDtypeStruct((N_ATOMS, F_PAD), jnp.float32),
        jax.ShapeDtypeStruct((N_EDGES, R_PAD), jnp.float32),
    ),
    mesh=_MESH,
    compiler_params=_PARAMS,
    scratch_types=[
        pltpu.VMEM((CA,), jnp.int32),
        pltpu.VMEM((CA, F_PAD), jnp.float32),
        pltpu.SemaphoreType.DMA,
        pltpu.VMEM((CE,), jnp.int32),
        pltpu.VMEM((CE,), jnp.int32),
        pltpu.VMEM((CE, R_PAD), jnp.float32),
        pltpu.VMEM((CE, R_PAD), jnp.float32),
        pltpu.VMEM((CE, R_PAD), jnp.float32),
        pltpu.SemaphoreType.DMA,
        pltpu.SemaphoreType.DMA,
        pltpu.SemaphoreType.DMA,
        pltpu.VMEM((CE,), jnp.int32),
        pltpu.VMEM((CE,), jnp.int32),
        pltpu.VMEM((CE, R_PAD), jnp.float32),
        pltpu.VMEM((CE, R_PAD), jnp.float32),
        pltpu.VMEM((CE, R_PAD), jnp.float32),
        pltpu.SemaphoreType.DMA,
        pltpu.SemaphoreType.DMA,
        pltpu.SemaphoreType.DMA,
    ],
)
def _embed_r_kernel(an_hbm, pos_hbm, src_hbm, dst_hbm, tab_hbm, feat_out, r_out,
                    aidx, afeat, asem,
                    sxa, dxa, psa, pda, rba, sia, sga, soa,
                    sxb, dxb, psb, pdb, rbb, sib, sgb, sob):
    wid = _wid()
    iota = lax.iota(jnp.int32, LANES)

    def atom_chunk(k, carry):
        c = wid + k * NW
        base = jnp.minimum(c * CA, N_ATOMS - CA)
        pltpu.sync_copy(an_hbm.at[pl.ds(base, CA)], aidx)
        pltpu.async_copy(tab_hbm.at[aidx], afeat, asem).wait()
        pltpu.sync_copy(afeat, feat_out.at[pl.ds(base, CA)])
        return carry

    lax.fori_loop(0, _my_chunks(NCH_A, wid), atom_chunk, 0)

    n = _my_chunks(NCH_E, wid)
    slots = ((sxa, dxa, psa, pda, rba, sia, sga, soa),
             (sxb, dxb, psb, pdb, rbb, sib, sgb, sob))

    def chunk_base(k):
        return jnp.minimum((wid + k * NW) * CE, N_EDGES - CE)

    def fire_idx(k, sl):
        base = chunk_base(k)
        pltpu.async_copy(src_hbm.at[pl.ds(base, CE)], sl[0], sl[5])
        pltpu.async_copy(dst_hbm.at[pl.ds(base, CE)], sl[1], sl[5])

    def wait_idx(sl):
        pltpu.make_async_copy(src_hbm.at[pl.ds(0, CE)], sl[0], sl[5]).wait()
        pltpu.make_async_copy(dst_hbm.at[pl.ds(0, CE)], sl[1], sl[5]).wait()

    def fire_gat(sl):
        pltpu.async_copy(pos_hbm.at[sl[0]], sl[2], sl[6])
        pltpu.async_copy(pos_hbm.at[sl[1]], sl[3], sl[6])

    def wait_gat(sl):
        pltpu.make_async_copy(pos_hbm.at[pl.ds(0, CE)], sl[2], sl[6]).wait()
        pltpu.make_async_copy(pos_hbm.at[pl.ds(0, CE)], sl[3], sl[6]).wait()

    def fire_out(k, sl):
        pltpu.async_copy(sl[4], r_out.at[pl.ds(chunk_base(k), CE)], sl[7])

    def wait_out(sl):
        pltpu.make_async_copy(sl[4], r_out.at[pl.ds(0, CE)], sl[7]).wait()

    def compute(sl):
        psrc, pdst, rbuf = sl[2], sl[3], sl[4]

        def sub_group(g, rows):
            for comp in range(3):
                cvec = jnp.full((LANES,), comp, jnp.int32)
                a = plsc.load_gather(pdst, [rows, cvec])
                b = plsc.load_gather(psrc, [rows, cvec])
                plsc.store_scatter(rbuf, [rows, cvec], a - b)
            return rows + LANES

        lax.fori_loop(0, CE // LANES, sub_group, iota)

    @pl.when(n >= 1)
    def _():
        fire_idx(0, slots[0])

    @pl.when(n >= 2)
    def _():
        fire_idx(1, slots[1])

    @pl.when(n >= 1)
    def _():
        wait_idx(slots[0])
        fire_gat(slots[0])

    def body(kk, carry):
        for off in (0, 1):
            k = kk * 2 + off
            s = slots[off]
            o = slots[1 - off]

            @pl.when(k < n)
            def _(k=k, s=s, o=o):
                wait_gat(s)

                @pl.when(k + 2 < n)
                def _():
                    fire_idx(k + 2, s)

                @pl.when(k + 1 < n)
                def _():
                    wait_idx(o)
                    fire_gat(o)

                @pl.when(k >= 2)
                def _():
                    wait_out(s)

                compute(s)
                fire_out(k, s)

        return carry

    lax.fori_loop(0, (n + 1) // 2, body, 0)

    last_even = lax.rem(n - 1, 2) == 0

    @pl.when(last_even)
    def _():
        wait_out(slots[0])

    @pl.when(jnp.logical_not(last_even))
    def _():
        wait_out(slots[1])

    @pl.when(n >= 2)
    def _():
        @pl.when(last_even)
        def _():
            wait_out(slots[1])

        @pl.when(jnp.logical_not(last_even))
        def _():
            wait_out(slots[0])


@functools.partial(
    pl.kernel,
    out_type=jax.ShapeDtypeStruct((N_PAIRS,), jnp.float32),
    mesh=_MESH,
    compiler_params=_PARAMS,
    scratch_types=[
        pltpu.VMEM((CP,), jnp.int32),
        pltpu.VMEM((CP,), jnp.int32),
        pltpu.VMEM((CP, R_PAD), jnp.float32),
        pltpu.VMEM((CP, R_PAD), jnp.float32),
        pltpu.VMEM((CP,), jnp.float32),
        pltpu.SemaphoreType.DMA,
        pltpu.SemaphoreType.DMA,
        pltpu.SemaphoreType.DMA,
        pltpu.VMEM((CP,), jnp.int32),
        pltpu.VMEM((CP,), jnp.int32),
        pltpu.VMEM((CP, R_PAD), jnp.float32),
        pltpu.VMEM((CP, R_PAD), jnp.float32),
        pltpu.VMEM((CP,), jnp.float32),
        pltpu.SemaphoreType.DMA,
        pltpu.SemaphoreType.DMA,
        pltpu.SemaphoreType.DMA,
        pltpu.VMEM((CP,), jnp.int32),
        pltpu.VMEM((CP,), jnp.int32),
        pltpu.VMEM((CP, R_PAD), jnp.float32),
        pltpu.VMEM((CP, R_PAD), jnp.float32),
        pltpu.VMEM((CP,), jnp.float32),
        pltpu.SemaphoreType.DMA,
        pltpu.SemaphoreType.DMA,
        pltpu.SemaphoreType.DMA,
    ],
)
def _cos_kernel(r_hbm, lg0_hbm, lg1_hbm, cos_out,
                i0a, i1a, r1a, r2a, cba, sia, sga, soa,
                i0b, i1b, r1b, r2b, cbb, sib, sgb, sob,
                i0c, i1c, r1c, r2c, cbc, sic, sgc, soc):
    wid = _wid()
    iota = lax.iota(jnp.int32, LANES)
    n = _my_chunks(NCH_P, wid)
    slots = ((i0a, i1a, r1a, r2a, cba, sia, sga, soa),
             (i0b, i1b, r1b, r2b, cbb, sib, sgb, sob),
             (i0c, i1c, r1c, r2c, cbc, sic, sgc, soc))

    def chunk_base(k):
        return jnp.minimum((wid + k * NW) * CP, N_PAIRS - CP)

    def fire_idx(k, sl):
        base = chunk_base(k)
        pltpu.async_copy(lg0_hbm.at[pl.ds(base, CP)], sl[0], sl[5])
        pltpu.async_copy(lg1_hbm.at[pl.ds(base, CP)], sl[1], sl[5])

    def wait_idx(sl):
        pltpu.make_async_copy(lg0_hbm.at[pl.ds(0, CP)], sl[0], sl[5]).wait()
        pltpu.make_async_copy(lg1_hbm.at[pl.ds(0, CP)], sl[1], sl[5]).wait()

    def fire_gat(sl):
        pltpu.async_copy(r_hbm.at[sl[0]], sl[2], sl[6])
        pltpu.async_copy(r_hbm.at[sl[1]], sl[3], sl[6])

    def wait_gat(sl):
        pltpu.make_async_copy(r_hbm.at[pl.ds(0, CP)], sl[2], sl[6]).wait()
        pltpu.make_async_copy(r_hbm.at[pl.ds(0, CP)], sl[3], sl[6]).wait()

    def fire_out(k, sl):
        pltpu.async_copy(sl[4], cos_out.at[pl.ds(chunk_base(k), CP)], sl[7])

    def wait_out(sl):
        pltpu.make_async_copy(sl[4], cos_out.at[pl.ds(0, CP)], sl[7]).wait()

    c0 = jnp.zeros((LANES,), jnp.int32)
    c1 = jnp.full((LANES,), 1, jnp.int32)
    c2v = jnp.full((LANES,), 2, jnp.int32)

    def compute(sl):
        r1, r2, cosb = sl[2], sl[3], sl[4]

        def grp(g, rows):
            x1 = plsc.load_gather(r1, [rows, c0])
            y1 = plsc.load_gather(r1, [rows, c1])
            z1 = plsc.load_gather(r1, [rows, c2v])
            x2 = plsc.load_gather(r2, [rows, c0])
            y2 = plsc.load_gather(r2, [rows, c1])
            z2 = plsc.load_gather(r2, [rows, c2v])
            num = x1 * x2 + y1 * y2 + z1 * z2
            q = (x1 * x1 + y1 * y1 + z1 * z1) * (x2 * x2 + y2 * y2 + z2 * z2)
            cosv = (0.0 - num) * _rsqrt(q)
            cosv = jnp.clip(cosv, -1.0, 1.0)
            cosb[pl.ds(g * LANES, LANES)] = cosv
            return rows + LANES

        lax.fori_loop(0, CP // LANES, grp, iota)

    @pl.when(n >= 1)
    def _():
        fire_idx(0, slots[0])

    @pl.when(n >= 2)
    def _():
        fire_idx(1, slots[1])

    @pl.when(n >= 3)
    def _():
        fire_idx(2, slots[2])

    @pl.when(n >= 1)
    def _():
        wait_idx(slots[0])
        fire_gat(slots[0])

    @pl.when(n >= 2)
    def _():
        wait_idx(slots[1])
        fire_gat(slots[1])

    def body(kk, carry):
        for off in (0, 1, 2):
            k = kk * 3 + off
            s = slots[off]
            nx = slots[(off + 2) % 3]

            @pl.when(k < n)
            def _(k=k, s=s, nx=nx):
                wait_gat(s)

                @pl.when(k + 3 < n)
                def _():
                    fire_idx(k + 3, s)

                @pl.when(k + 2 < n)
                def _():
                    wait_idx(nx)
                    fire_gat(nx)

                @pl.when(k >= 3)
                def _():
                    wait_out(s)

                compute(s)
                fire_out(k, s)

        return carry

    lax.fori_loop(0, (n + 2) // 3, body, 0)

    for j in (1, 2, 3):
        for res in (0, 1, 2):
            @pl.when((n >= j) & (lax.rem(n - j, 3) == res))
            def _(res=res):
                wait_out(slots[res])


def kernel(atomic_number, positions, edge_index, lg_pairs, atom_table):
    an = atomic_number.astype(jnp.int32)
    ei = edge_index.astype(jnp.int32)
    lg = lg_pairs.astype(jnp.int32)
    src_e, dst_e = ei[0], ei[1]
    lg0, lg1 = lg[0], lg[1]
    pos_pad = jnp.pad(positions, ((0, 0), (0, R_PAD - 3)))
    tab_pad = jnp.pad(atom_table, ((0, 0), (0, F_PAD - F_DIM)))
    featp, r_pad = _embed_r_kernel(an, pos_pad, src_e, dst_e, tab_pad)
    cos = _cos_kernel(r_pad, lg0, lg1)
    return (featp[:, :F_DIM] * 1.0, r_pad[:, :3] * 1.0, cos)


# speedup vs baseline: 1.0281x; 1.0281x over previous
import functools

import jax
import jax.numpy as jnp
from jax import lax
from jax.experimental import pallas as pl
from jax.experimental.pallas import tpu as pltpu
from jax.experimental.pallas import tpu_sc as plsc

N_ATOMS = 50000
N_EDGES = 800000
N_PAIRS = 3200000
F_DIM = 92
F_PAD = 96
R_PAD = 8

NC, NS, LANES = 2, 16, 16
NW = NC * NS

CA = 512
CE = 1024
CP = 2048


def _cdiv(a, b):
    return (a + b - 1) // b


NCH_A = _cdiv(N_ATOMS, CA)
NCH_E = _cdiv(N_EDGES, CE)
NCH_P = _cdiv(N_PAIRS, CP)

_MESH = plsc.VectorSubcoreMesh(
    core_axis_name="c", subcore_axis_name="s", num_cores=NC, num_subcores=NS
)
_PARAMS = pltpu.CompilerParams(
    needs_layout_passes=False, use_tc_tiling_on_sc=False
)


def _wid():
    return lax.axis_index("s") * NC + lax.axis_index("c")


def _my_chunks(total_chunks, wid):
    return (total_chunks - 1 - wid) // NW + 1


def _rsqrt(q):
    i = plsc.bitcast(q, jnp.int32)
    i = 0x5F3759DF - lax.shift_right_logical(i, 1)
    y = plsc.bitcast(i, jnp.float32)
    for _ in range(2):
        y = y * (1.5 - 0.5 * q * y * y)
    return y


@functools.partial(
    pl.kernel,
    out_type=(
        jax.ShapeDtypeStruct((N_ATOMS, F_PAD), jnp.float32),
        jax.ShapeDtypeStruct((N_EDGES, R_PAD), jnp.float32),
    ),
    mesh=_MESH,
    compiler_params=_PARAMS,
    scratch_types=[
        pltpu.VMEM((CA,), jnp.int32),
        pltpu.VMEM((CA, F_PAD), jnp.float32),
        pltpu.SemaphoreType.DMA,
        pltpu.VMEM((CE,), jnp.int32),
        pltpu.VMEM((CE,), jnp.int32),
        pltpu.VMEM((CE, R_PAD), jnp.float32),
        pltpu.VMEM((CE, R_PAD), jnp.float32),
        pltpu.VMEM((CE, R_PAD), jnp.float32),
        pltpu.SemaphoreType.DMA,
        pltpu.SemaphoreType.DMA,
        pltpu.SemaphoreType.DMA,
        pltpu.VMEM((CE,), jnp.int32),
        pltpu.VMEM((CE,), jnp.int32),
        pltpu.VMEM((CE, R_PAD), jnp.float32),
        pltpu.VMEM((CE, R_PAD), jnp.float32),
        pltpu.VMEM((CE, R_PAD), jnp.float32),
        pltpu.SemaphoreType.DMA,
        pltpu.SemaphoreType.DMA,
        pltpu.SemaphoreType.DMA,
    ],
)
def _embed_r_kernel(an_hbm, pos_hbm, src_hbm, dst_hbm, tab_hbm, feat_out, r_out,
                    aidx, afeat, asem,
                    sxa, dxa, psa, pda, rba, sia, sga, soa,
                    sxb, dxb, psb, pdb, rbb, sib, sgb, sob):
    wid = _wid()
    iota = lax.iota(jnp.int32, LANES)

    def atom_chunk(k, carry):
        c = wid + k * NW
        base = jnp.minimum(c * CA, N_ATOMS - CA)
        pltpu.sync_copy(an_hbm.at[pl.ds(base, CA)], aidx)
        pltpu.async_copy(tab_hbm.at[aidx], afeat, asem).wait()
        pltpu.sync_copy(afeat, feat_out.at[pl.ds(base, CA)])
        return carry

    lax.fori_loop(0, _my_chunks(NCH_A, wid), atom_chunk, 0)

    n = _my_chunks(NCH_E, wid)
    slots = ((sxa, dxa, psa, pda, rba, sia, sga, soa),
             (sxb, dxb, psb, pdb, rbb, sib, sgb, sob))

    def chunk_base(k):
        return jnp.minimum((wid + k * NW) * CE, N_EDGES - CE)

    def fire_idx(k, sl):
        base = chunk_base(k)
        pltpu.async_copy(src_hbm.at[pl.ds(base, CE)], sl[0], sl[5])
        pltpu.async_copy(dst_hbm.at[pl.ds(base, CE)], sl[1], sl[5])

    def wait_idx(sl):
        pltpu.make_async_copy(src_hbm.at[pl.ds(0, CE)], sl[0], sl[5]).wait()
        pltpu.make_async_copy(dst_hbm.at[pl.ds(0, CE)], sl[1], sl[5]).wait()

    def fire_gat(sl):
        pltpu.async_copy(pos_hbm.at[sl[0]], sl[2], sl[6])
        pltpu.async_copy(pos_hbm.at[sl[1]], sl[3], sl[6])

    def wait_gat(sl):
        pltpu.make_async_copy(pos_hbm.at[pl.ds(0, CE)], sl[2], sl[6]).wait()
        pltpu.make_async_copy(pos_hbm.at[pl.ds(0, CE)], sl[3], sl[6]).wait()

    def fire_out(k, sl):
        pltpu.async_copy(sl[4], r_out.at[pl.ds(chunk_base(k), CE)], sl[7])

    def wait_out(sl):
        pltpu.make_async_copy(sl[4], r_out.at[pl.ds(0, CE)], sl[7]).wait()

    def compute(sl):
        psrc, pdst, rbuf = sl[2], sl[3], sl[4]

        def sub_group(g, rows):
            for comp in range(3):
                cvec = jnp.full((LANES,), comp, jnp.int32)
                a = plsc.load_gather(pdst, [rows, cvec])
                b = plsc.load_gather(psrc, [rows, cvec])
                plsc.store_scatter(rbuf, [rows, cvec], a - b)
            return rows + LANES

        lax.fori_loop(0, CE // LANES, sub_group, iota)

    @pl.when(n >= 1)
    def _():
        fire_idx(0, slots[0])

    @pl.when(n >= 2)
    def _():
        fire_idx(1, slots[1])

    @pl.when(n >= 1)
    def _():
        wait_idx(slots[0])
        fire_gat(slots[0])

    def body(kk, carry):
        for off in (0, 1):
            k = kk * 2 + off
            s = slots[off]
            o = slots[1 - off]

            @pl.when(k < n)
            def _(k=k, s=s, o=o):
                wait_gat(s)

                @pl.when(k + 2 < n)
                def _():
                    fire_idx(k + 2, s)

                @pl.when(k + 1 < n)
                def _():
                    wait_idx(o)
                    fire_gat(o)

                @pl.when(k >= 2)
                def _():
                    wait_out(s)

                compute(s)
                fire_out(k, s)

        return carry

    lax.fori_loop(0, (n + 1) // 2, body, 0)

    last_even = lax.rem(n - 1, 2) == 0

    @pl.when(last_even)
    def _():
        wait_out(slots[0])

    @pl.when(jnp.logical_not(last_even))
    def _():
        wait_out(slots[1])

    @pl.when(n >= 2)
    def _():
        @pl.when(last_even)
        def _():
            wait_out(slots[1])

        @pl.when(jnp.logical_not(last_even))
        def _():
            wait_out(slots[0])


@functools.partial(
    pl.kernel,
    out_type=jax.ShapeDtypeStruct((N_PAIRS,), jnp.float32),
    mesh=_MESH,
    compiler_params=_PARAMS,
    scratch_types=[
        pltpu.VMEM((CP,), jnp.int32),
        pltpu.VMEM((CP,), jnp.int32),
        pltpu.VMEM((CP, R_PAD), jnp.float32),
        pltpu.VMEM((CP, R_PAD), jnp.float32),
        pltpu.VMEM((CP,), jnp.float32),
        pltpu.SemaphoreType.DMA,
        pltpu.SemaphoreType.DMA,
        pltpu.SemaphoreType.DMA,
        pltpu.VMEM((CP,), jnp.int32),
        pltpu.VMEM((CP,), jnp.int32),
        pltpu.VMEM((CP, R_PAD), jnp.float32),
        pltpu.VMEM((CP, R_PAD), jnp.float32),
        pltpu.VMEM((CP,), jnp.float32),
        pltpu.SemaphoreType.DMA,
        pltpu.SemaphoreType.DMA,
        pltpu.SemaphoreType.DMA,
        pltpu.VMEM((CP,), jnp.int32),
        pltpu.VMEM((CP,), jnp.int32),
        pltpu.VMEM((CP, R_PAD), jnp.float32),
        pltpu.VMEM((CP, R_PAD), jnp.float32),
        pltpu.VMEM((CP,), jnp.float32),
        pltpu.SemaphoreType.DMA,
        pltpu.SemaphoreType.DMA,
        pltpu.SemaphoreType.DMA,
    ],
)
def _cos_kernel(r_hbm, lg0_hbm, lg1_hbm, cos_out,
                i0a, i1a, r1a, r2a, cba, sia, sga, soa,
                i0b, i1b, r1b, r2b, cbb, sib, sgb, sob,
                i0c, i1c, r1c, r2c, cbc, sic, sgc, soc):
    wid = _wid()
    iota = lax.iota(jnp.int32, LANES)
    n = _my_chunks(NCH_P, wid)
    slots = ((i0a, i1a, r1a, r2a, cba, sia, sga, soa),
             (i0b, i1b, r1b, r2b, cbb, sib, sgb, sob),
             (i0c, i1c, r1c, r2c, cbc, sic, sgc, soc))

    def chunk_base(k):
        return jnp.minimum((wid + k * NW) * CP, N_PAIRS - CP)

    def fire_idx(k, sl):
        base = chunk_base(k)
        pltpu.async_copy(lg0_hbm.at[pl.ds(base, CP)], sl[0], sl[5])
        pltpu.async_copy(lg1_hbm.at[pl.ds(base, CP)], sl[1], sl[5])

    def wait_idx(sl):
        pltpu.make_async_copy(lg0_hbm.at[pl.ds(0, CP)], sl[0], sl[5]).wait()
        pltpu.make_async_copy(lg1_hbm.at[pl.ds(0, CP)], sl[1], sl[5]).wait()

    def fire_gat(sl):
        pltpu.async_copy(r_hbm.at[sl[0]], sl[2], sl[6])
        pltpu.async_copy(r_hbm.at[sl[1]], sl[3], sl[6])

    def wait_gat(sl):
        pltpu.make_async_copy(r_hbm.at[pl.ds(0, CP)], sl[2], sl[6]).wait()
        pltpu.make_async_copy(r_hbm.at[pl.ds(0, CP)], sl[3], sl[6]).wait()

    def fire_out(k, sl):
        pltpu.async_copy(sl[4], cos_out.at[pl.ds(chunk_base(k), CP)], sl[7])

    def wait_out(sl):
        pltpu.make_async_copy(sl[4], cos_out.at[pl.ds(0, CP)], sl[7]).wait()

    c0 = jnp.zeros((LANES,), jnp.int32)
    c1 = jnp.full((LANES,), 1, jnp.int32)
    c2v = jnp.full((LANES,), 2, jnp.int32)

    def compute(sl):
        r1, r2, cosb = sl[2], sl[3], sl[4]

        def grp(g, rows):
            x1 = plsc.load_gather(r1, [rows, c0])
            y1 = plsc.load_gather(r1, [rows, c1])
            z1 = plsc.load_gather(r1, [rows, c2v])
            x2 = plsc.load_gather(r2, [rows, c0])
            y2 = plsc.load_gather(r2, [rows, c1])
            z2 = plsc.load_gather(r2, [rows, c2v])
            num = x1 * x2 + y1 * y2 + z1 * z2
            q = (x1 * x1 + y1 * y1 + z1 * z1) * (x2 * x2 + y2 * y2 + z2 * z2)
            cosv = (0.0 - num) * _rsqrt(q)
            cosv = jnp.clip(cosv, -1.0, 1.0)
            cosb[pl.ds(g * LANES, LANES)] = cosv
            return rows + LANES

        lax.fori_loop(0, CP // LANES, grp, iota)

    @pl.when(n >= 1)
    def _():
        fire_idx(0, slots[0])

    @pl.when(n >= 2)
    def _():
        fire_idx(1, slots[1])

    @pl.when(n >= 3)
    def _():
        fire_idx(2, slots[2])

    @pl.when(n >= 1)
    def _():
        wait_idx(slots[0])
        fire_gat(slots[0])

    @pl.when(n >= 2)
    def _():
        wait_idx(slots[1])
        fire_gat(slots[1])

    def body(kk, carry):
        for off in (0, 1, 2):
            k = kk * 3 + off
            s = slots[off]
            nx = slots[(off + 2) % 3]

            @pl.when(k < n)
            def _(k=k, s=s, nx=nx):
                wait_gat(s)

                @pl.when(k + 3 < n)
                def _():
                    fire_idx(k + 3, s)

                @pl.when(k + 2 < n)
                def _():
                    wait_idx(nx)
                    fire_gat(nx)

                @pl.when(k >= 3)
                def _():
                    wait_out(s)

                compute(s)
                fire_out(k, s)

        return carry

    lax.fori_loop(0, (n + 2) // 3, body, 0)

    for j in (1, 2, 3):
        for res in (0, 1, 2):
            @pl.when((n >= j) & (lax.rem(n - j, 3) == res))
            def _(res=res):
                wait_out(slots[res])

_G1 = 25
_EB = 32768
_LB = 131072
_NB = 2000


@functools.partial(
    pl.pallas_call,
    grid=(_G1,),
    in_specs=[
        pl.BlockSpec((2, _EB), lambda i: (0, i)),
        pl.BlockSpec((2, _LB), lambda i: (0, i)),
    ],
    out_specs=[
        pl.BlockSpec((_EB,), lambda i: (i,)),
        pl.BlockSpec((_EB,), lambda i: (i,)),
        pl.BlockSpec((_LB,), lambda i: (i,)),
        pl.BlockSpec((_LB,), lambda i: (i,)),
    ],
    out_shape=[
        jax.ShapeDtypeStruct((N_EDGES,), jnp.int32),
        jax.ShapeDtypeStruct((N_EDGES,), jnp.int32),
        jax.ShapeDtypeStruct((N_PAIRS,), jnp.int32),
        jax.ShapeDtypeStruct((N_PAIRS,), jnp.int32),
    ],
)
def _prep_tc(ei_ref, lg_ref, src_ref, dst_ref, lg0_ref, lg1_ref):
    src_ref[...] = ei_ref[0, :]
    dst_ref[...] = ei_ref[1, :]
    lg0_ref[...] = lg_ref[0, :]
    lg1_ref[...] = lg_ref[1, :]


def kernel(atomic_number, positions, edge_index, lg_pairs, atom_table):
    an = atomic_number.astype(jnp.int32)
    ei = edge_index.astype(jnp.int32)
    lg = lg_pairs.astype(jnp.int32)
    src_e, dst_e, lg0, lg1 = _prep_tc(ei, lg)
    pos_pad = jnp.pad(positions, ((0, 0), (0, R_PAD - 3)))
    tab_pad = jnp.pad(atom_table, ((0, 0), (0, F_PAD - F_DIM)))
    featp, r_pad = _embed_r_kernel(an, pos_pad, src_e, dst_e, tab_pad)
    cos = _cos_kernel(r_pad, lg0, lg1)
    return (featp[:, :F_DIM], r_pad[:, :3], cos)


# speedup vs baseline: 1.0430x; 1.0145x over previous
import functools

import jax
import jax.numpy as jnp
from jax import lax
from jax.experimental import pallas as pl
from jax.experimental.pallas import tpu as pltpu
from jax.experimental.pallas import tpu_sc as plsc

N_ATOMS = 50000
N_EDGES = 800000
N_PAIRS = 3200000
F_DIM = 92
F_PAD = 96
R_PAD = 8

NC, NS, LANES = 2, 16, 16
NW = NC * NS

CA = 512
CE = 1024
CP = 2048


def _cdiv(a, b):
    return (a + b - 1) // b


NCH_A = _cdiv(N_ATOMS, CA)
NCH_E = _cdiv(N_EDGES, CE)
NCH_P = _cdiv(N_PAIRS, CP)

_MESH = plsc.VectorSubcoreMesh(
    core_axis_name="c", subcore_axis_name="s", num_cores=NC, num_subcores=NS
)
_PARAMS = pltpu.CompilerParams(
    needs_layout_passes=False, use_tc_tiling_on_sc=False
)


def _wid():
    return lax.axis_index("s") * NC + lax.axis_index("c")


def _my_chunks(total_chunks, wid):
    return (total_chunks - 1 - wid) // NW + 1


def _rsqrt(q):
    i = plsc.bitcast(q, jnp.int32)
    i = 0x5F3759DF - lax.shift_right_logical(i, 1)
    y = plsc.bitcast(i, jnp.float32)
    for _ in range(2):
        y = y * (1.5 - 0.5 * q * y * y)
    return y


@functools.partial(
    pl.kernel,
    out_type=(
        jax.ShapeDtypeStruct((N_ATOMS, F_PAD), jnp.float32),
        jax.ShapeDtypeStruct((N_EDGES, R_PAD), jnp.float32),
    ),
    mesh=_MESH,
    compiler_params=_PARAMS,
    scratch_types=[
        pltpu.VMEM((CA,), jnp.int32),
        pltpu.VMEM((CA, F_PAD), jnp.float32),
        pltpu.SemaphoreType.DMA,
        pltpu.VMEM((CE,), jnp.int32),
        pltpu.VMEM((CE,), jnp.int32),
        pltpu.VMEM((CE, R_PAD), jnp.float32),
        pltpu.VMEM((CE, R_PAD), jnp.float32),
        pltpu.VMEM((CE, R_PAD), jnp.float32),
        pltpu.SemaphoreType.DMA,
        pltpu.SemaphoreType.DMA,
        pltpu.SemaphoreType.DMA,
        pltpu.VMEM((CE,), jnp.int32),
        pltpu.VMEM((CE,), jnp.int32),
        pltpu.VMEM((CE, R_PAD), jnp.float32),
        pltpu.VMEM((CE, R_PAD), jnp.float32),
        pltpu.VMEM((CE, R_PAD), jnp.float32),
        pltpu.SemaphoreType.DMA,
        pltpu.SemaphoreType.DMA,
        pltpu.SemaphoreType.DMA,
    ],
)
def _embed_r_kernel(an_hbm, pos_hbm, src_hbm, dst_hbm, tab_hbm, feat_out, r_out,
                    aidx, afeat, asem,
                    sxa, dxa, psa, pda, rba, sia, sga, soa,
                    sxb, dxb, psb, pdb, rbb, sib, sgb, sob):
    wid = _wid()
    iota = lax.iota(jnp.int32, LANES)

    def atom_chunk(k, carry):
        c = wid + k * NW
        base = jnp.minimum(c * CA, N_ATOMS - CA)
        pltpu.sync_copy(an_hbm.at[pl.ds(base, CA)], aidx)
        pltpu.async_copy(tab_hbm.at[aidx], afeat, asem).wait()
        pltpu.sync_copy(afeat, feat_out.at[pl.ds(base, CA)])
        return carry

    lax.fori_loop(0, _my_chunks(NCH_A, wid), atom_chunk, 0)

    n = _my_chunks(NCH_E, wid)
    slots = ((sxa, dxa, psa, pda, rba, sia, sga, soa),
             (sxb, dxb, psb, pdb, rbb, sib, sgb, sob))

    def chunk_base(k):
        return jnp.minimum((wid + k * NW) * CE, N_EDGES - CE)

    def fire_idx(k, sl):
        base = chunk_base(k)
        pltpu.async_copy(src_hbm.at[pl.ds(base, CE)], sl[0], sl[5])
        pltpu.async_copy(dst_hbm.at[pl.ds(base, CE)], sl[1], sl[5])

    def wait_idx(sl):
        pltpu.make_async_copy(src_hbm.at[pl.ds(0, CE)], sl[0], sl[5]).wait()
        pltpu.make_async_copy(dst_hbm.at[pl.ds(0, CE)], sl[1], sl[5]).wait()

    def fire_gat(sl):
        pltpu.async_copy(pos_hbm.at[sl[0]], sl[2], sl[6])
        pltpu.async_copy(pos_hbm.at[sl[1]], sl[3], sl[6])

    def wait_gat(sl):
        pltpu.make_async_copy(pos_hbm.at[pl.ds(0, CE)], sl[2], sl[6]).wait()
        pltpu.make_async_copy(pos_hbm.at[pl.ds(0, CE)], sl[3], sl[6]).wait()

    def fire_out(k, sl):
        pltpu.async_copy(sl[4], r_out.at[pl.ds(chunk_base(k), CE)], sl[7])

    def wait_out(sl):
        pltpu.make_async_copy(sl[4], r_out.at[pl.ds(0, CE)], sl[7]).wait()

    def compute(sl):
        psrc, pdst, rbuf = sl[2], sl[3], sl[4]

        def sub_group(g, rows):
            for comp in range(3):
                cvec = jnp.full((LANES,), comp, jnp.int32)
                a = plsc.load_gather(pdst, [rows, cvec])
                b = plsc.load_gather(psrc, [rows, cvec])
                plsc.store_scatter(rbuf, [rows, cvec], a - b)
            return rows + LANES

        lax.fori_loop(0, CE // LANES, sub_group, iota)

    @pl.when(n >= 1)
    def _():
        fire_idx(0, slots[0])

    @pl.when(n >= 2)
    def _():
        fire_idx(1, slots[1])

    @pl.when(n >= 1)
    def _():
        wait_idx(slots[0])
        fire_gat(slots[0])

    def body(kk, carry):
        for off in (0, 1):
            k = kk * 2 + off
            s = slots[off]
            o = slots[1 - off]

            @pl.when(k < n)
            def _(k=k, s=s, o=o):
                wait_gat(s)

                @pl.when(k + 2 < n)
                def _():
                    fire_idx(k + 2, s)

                @pl.when(k + 1 < n)
                def _():
                    wait_idx(o)
                    fire_gat(o)

                @pl.when(k >= 2)
                def _():
                    wait_out(s)

                compute(s)
                fire_out(k, s)

        return carry

    lax.fori_loop(0, (n + 1) // 2, body, 0)

    last_even = lax.rem(n - 1, 2) == 0

    @pl.when(last_even)
    def _():
        wait_out(slots[0])

    @pl.when(jnp.logical_not(last_even))
    def _():
        wait_out(slots[1])

    @pl.when(n >= 2)
    def _():
        @pl.when(last_even)
        def _():
            wait_out(slots[1])

        @pl.when(jnp.logical_not(last_even))
        def _():
            wait_out(slots[0])


@functools.partial(
    pl.kernel,
    out_type=jax.ShapeDtypeStruct((N_PAIRS,), jnp.float32),
    mesh=_MESH,
    compiler_params=_PARAMS,
    scratch_types=[
        pltpu.VMEM((CP,), jnp.int32),
        pltpu.VMEM((CP,), jnp.int32),
        pltpu.VMEM((CP, R_PAD), jnp.float32),
        pltpu.VMEM((CP, R_PAD), jnp.float32),
        pltpu.VMEM((CP,), jnp.float32),
        pltpu.SemaphoreType.DMA,
        pltpu.SemaphoreType.DMA,
        pltpu.SemaphoreType.DMA,
        pltpu.VMEM((CP,), jnp.int32),
        pltpu.VMEM((CP,), jnp.int32),
        pltpu.VMEM((CP, R_PAD), jnp.float32),
        pltpu.VMEM((CP, R_PAD), jnp.float32),
        pltpu.VMEM((CP,), jnp.float32),
        pltpu.SemaphoreType.DMA,
        pltpu.SemaphoreType.DMA,
        pltpu.SemaphoreType.DMA,
        pltpu.VMEM((CP,), jnp.int32),
        pltpu.VMEM((CP,), jnp.int32),
        pltpu.VMEM((CP, R_PAD), jnp.float32),
        pltpu.VMEM((CP, R_PAD), jnp.float32),
        pltpu.VMEM((CP,), jnp.float32),
        pltpu.SemaphoreType.DMA,
        pltpu.SemaphoreType.DMA,
        pltpu.SemaphoreType.DMA,
    ],
)
def _cos_kernel(r_hbm, lg0_hbm, lg1_hbm, cos_out,
                i0a, i1a, r1a, r2a, cba, sia, sga, soa,
                i0b, i1b, r1b, r2b, cbb, sib, sgb, sob,
                i0c, i1c, r1c, r2c, cbc, sic, sgc, soc):
    wid = _wid()
    iota = lax.iota(jnp.int32, LANES)
    n = _my_chunks(NCH_P, wid)
    slots = ((i0a, i1a, r1a, r2a, cba, sia, sga, soa),
             (i0b, i1b, r1b, r2b, cbb, sib, sgb, sob),
             (i0c, i1c, r1c, r2c, cbc, sic, sgc, soc))

    def chunk_base(k):
        return jnp.minimum((wid + k * NW) * CP, N_PAIRS - CP)

    def fire_idx(k, sl):
        base = chunk_base(k)
        pltpu.async_copy(lg0_hbm.at[pl.ds(base, CP)], sl[0], sl[5])
        pltpu.async_copy(lg1_hbm.at[pl.ds(base, CP)], sl[1], sl[5])

    def wait_idx(sl):
        pltpu.make_async_copy(lg0_hbm.at[pl.ds(0, CP)], sl[0], sl[5]).wait()
        pltpu.make_async_copy(lg1_hbm.at[pl.ds(0, CP)], sl[1], sl[5]).wait()

    def fire_gat(sl):
        pltpu.async_copy(r_hbm.at[sl[0]], sl[2], sl[6])
        pltpu.async_copy(r_hbm.at[sl[1]], sl[3], sl[6])

    def wait_gat(sl):
        pltpu.make_async_copy(r_hbm.at[pl.ds(0, CP)], sl[2], sl[6]).wait()
        pltpu.make_async_copy(r_hbm.at[pl.ds(0, CP)], sl[3], sl[6]).wait()

    def fire_out(k, sl):
        pltpu.async_copy(sl[4], cos_out.at[pl.ds(chunk_base(k), CP)], sl[7])

    def wait_out(sl):
        pltpu.make_async_copy(sl[4], cos_out.at[pl.ds(0, CP)], sl[7]).wait()

    c0 = jnp.zeros((LANES,), jnp.int32)
    c1 = jnp.full((LANES,), 1, jnp.int32)
    c2v = jnp.full((LANES,), 2, jnp.int32)

    def compute(sl):
        r1, r2, cosb = sl[2], sl[3], sl[4]

        def grp(g, rows):
            x1 = plsc.load_gather(r1, [rows, c0])
            y1 = plsc.load_gather(r1, [rows, c1])
            z1 = plsc.load_gather(r1, [rows, c2v])
            x2 = plsc.load_gather(r2, [rows, c0])
            y2 = plsc.load_gather(r2, [rows, c1])
            z2 = plsc.load_gather(r2, [rows, c2v])
            num = x1 * x2 + y1 * y2 + z1 * z2
            q = (x1 * x1 + y1 * y1 + z1 * z1) * (x2 * x2 + y2 * y2 + z2 * z2)
            cosv = (0.0 - num) * _rsqrt(q)
            cosv = jnp.clip(cosv, -1.0, 1.0)
            cosb[pl.ds(g * LANES, LANES)] = cosv
            return rows + LANES

        lax.fori_loop(0, CP // LANES, grp, iota)

    @pl.when(n >= 1)
    def _():
        fire_idx(0, slots[0])

    @pl.when(n >= 2)
    def _():
        fire_idx(1, slots[1])

    @pl.when(n >= 3)
    def _():
        fire_idx(2, slots[2])

    @pl.when(n >= 1)
    def _():
        wait_idx(slots[0])
        fire_gat(slots[0])

    @pl.when(n >= 2)
    def _():
        wait_idx(slots[1])
        fire_gat(slots[1])

    def body(kk, carry):
        for off in (0, 1, 2):
            k = kk * 3 + off
            s = slots[off]
            nx = slots[(off + 2) % 3]

            @pl.when(k < n)
            def _(k=k, s=s, nx=nx):
                wait_gat(s)

                @pl.when(k + 3 < n)
                def _():
                    fire_idx(k + 3, s)

                @pl.when(k + 2 < n)
                def _():
                    wait_idx(nx)
                    fire_gat(nx)

                @pl.when(k >= 3)
                def _():
                    wait_out(s)

                compute(s)
                fire_out(k, s)

        return carry

    lax.fori_loop(0, (n + 2) // 3, body, 0)

    for j in (1, 2, 3):
        for res in (0, 1, 2):
            @pl.when((n >= j) & (lax.rem(n - j, 3) == res))
            def _(res=res):
                wait_out(slots[res])

_G1 = 25
_EB = 32768
_LB = 131072
_NB = 2000


@functools.partial(
    pl.pallas_call,
    grid=(_G1,),
    in_specs=[
        pl.BlockSpec((2, _EB), lambda i: (0, i)),
        pl.BlockSpec((2, _LB), lambda i: (0, i)),
    ],
    out_specs=[
        pl.BlockSpec((_EB,), lambda i: (i,)),
        pl.BlockSpec((_EB,), lambda i: (i,)),
        pl.BlockSpec((_LB,), lambda i: (i,)),
        pl.BlockSpec((_LB,), lambda i: (i,)),
    ],
    out_shape=[
        jax.ShapeDtypeStruct((N_EDGES,), jnp.int32),
        jax.ShapeDtypeStruct((N_EDGES,), jnp.int32),
        jax.ShapeDtypeStruct((N_PAIRS,), jnp.int32),
        jax.ShapeDtypeStruct((N_PAIRS,), jnp.int32),
    ],
)
def _prep_tc(ei_ref, lg_ref, src_ref, dst_ref, lg0_ref, lg1_ref):
    src_ref[...] = ei_ref[0, :]
    dst_ref[...] = ei_ref[1, :]
    lg0_ref[...] = lg_ref[0, :]
    lg1_ref[...] = lg_ref[1, :]


@functools.partial(
    pl.pallas_call,
    grid=(_G1,),
    in_specs=[pl.BlockSpec((_NB, F_PAD), lambda i: (i, 0))],
    out_specs=pl.BlockSpec((_NB, F_DIM), lambda i: (i, 0)),
    out_shape=jax.ShapeDtypeStruct((N_ATOMS, F_DIM), jnp.float32),
)
def _depad_feat_tc(featp_ref, feat_ref):
    feat_ref[...] = featp_ref[:, :F_DIM]


def kernel(atomic_number, positions, edge_index, lg_pairs, atom_table):
    an = atomic_number.astype(jnp.int32)
    ei = edge_index.astype(jnp.int32)
    lg = lg_pairs.astype(jnp.int32)
    src_e, dst_e, lg0, lg1 = _prep_tc(ei, lg)
    pos_pad = jnp.pad(positions, ((0, 0), (0, R_PAD - 3)))
    tab_pad = jnp.pad(atom_table, ((0, 0), (0, F_PAD - F_DIM)))
    featp, r_pad = _embed_r_kernel(an, pos_pad, src_e, dst_e, tab_pad)
    cos = _cos_kernel(r_pad, lg0, lg1)
    return (_depad_feat_tc(featp), r_pad[:, :3], cos)
